# baseline (speedup 1.0000x reference)
import functools

import jax
import jax.numpy as jnp
from jax import lax
from jax.experimental import pallas as pl
from jax.experimental.pallas import tpu as pltpu
from jax.experimental.pallas import tpu_sc as plsc

S = 2048
H = 768
NH = 12
HD = 64
NR = 14
NSH = 2
TOPK = 2
INTER = 768
BLK = 128
NBLK = 46
PAD = NBLK * BLK
NE = TOPK * S
F32 = jnp.float32
I32 = jnp.int32
_SCALE = 1.0 / (HD ** 0.5)
_SQRT_HALF = 0.7071067811865476


def _ln(h, g, b):
    m = jnp.mean(h, axis=-1, keepdims=True)
    c = h - m
    v = jnp.mean(c * c, axis=-1, keepdims=True)
    return c * lax.rsqrt(v + 1e-5) * g + b


def _gelu(x):
    return 0.5 * x * (1.0 + lax.erf(x * _SQRT_HALF))


_QB = 256
_NQ = S // _QB


def _attn_body(x_ref, g_ref, b_ref, wq_ref, bq_ref, wk_ref, bk_ref,
               wv_ref, bv_ref, o_ref, q_s, k_s, v_s):
    j = pl.program_id(0)

    @pl.when(j < _NQ)
    def _():
        h = _ln(x_ref[...], g_ref[...], b_ref[...])
        i = j * _QB
        q_s[pl.ds(i, _QB), :] = (jnp.dot(h, wq_ref[...],
                                         preferred_element_type=F32)
                                 + bq_ref[...]) * _SCALE
        k_s[pl.ds(i, _QB), :] = (jnp.dot(h, wk_ref[...],
                                         preferred_element_type=F32)
                                 + bk_ref[...])
        v_s[pl.ds(i, _QB), :] = (jnp.dot(h, wv_ref[...],
                                         preferred_element_type=F32)
                                 + bv_ref[...])

    @pl.when(j >= _NQ)
    def _():
        hp = j - _NQ
        c0 = pl.multiple_of(hp * 2 * HD, 2 * HD)
        qp = q_s[:, pl.ds(c0, 2 * HD)]
        kp = k_s[:, pl.ds(c0, 2 * HD)]
        vp = v_s[:, pl.ds(c0, 2 * HD)]
        for t in range(2):
            q = qp[:, t * HD:(t + 1) * HD]
            k = kp[:, t * HD:(t + 1) * HD]
            v = vp[:, t * HD:(t + 1) * HD]
            s = lax.dot_general(q, k, (((1,), (1,)), ((), ())),
                                preferred_element_type=F32)
            p = jnp.exp(s)
            ve = jnp.concatenate([v, jnp.ones((S, 1), F32)], axis=1)
            r = jnp.dot(p, ve, preferred_element_type=F32)
            o_ref[:, t * HD:(t + 1) * HD] = r[:, :HD] / r[:, HD:HD + 1]


def _attn_call(x, g, b, wq, bq, wk, bk, wv, bv):
    hw = 2 * HD
    full = pl.BlockSpec((H, H), lambda j: (0, 0))
    vec = pl.BlockSpec((1, H), lambda j: (0, 0))
    xrow = pl.BlockSpec((_QB, H), lambda j: (jnp.minimum(j, _NQ - 1), 0))
    return pl.pallas_call(
        _attn_body,
        grid=(_NQ + NH // 2,),
        in_specs=[xrow, vec, vec, full, vec, full, vec, full, vec],
        out_specs=pl.BlockSpec((S, hw),
                               lambda j: (0, jnp.maximum(j - _NQ, 0))),
        out_shape=jax.ShapeDtypeStruct((S, H), F32),
        scratch_shapes=[pltpu.VMEM((S, H), F32)] * 3,
    )(x, g, b, wq, bq, wk, bk, wv, bv)


_PB = 512
_NP = S // _PB


def _post_body(ctx_ref, wo_ref, bo_ref, x_ref, g2_ref, b2_ref, wr_ref, br_ref,
               a_ref, h2_ref, tv_ref, dest_ref, be_ref, aff_s):
    j = pl.program_id(0)

    @pl.when(j < _NP)
    def _():
        a = jnp.dot(ctx_ref[...], wo_ref[...], preferred_element_type=F32)
        a = a + bo_ref[...] + x_ref[...]
        a_ref[...] = a
        h2 = _ln(a, g2_ref[...], b2_ref[...])
        h2_ref[...] = h2
        logits = (jnp.dot(h2, wr_ref[...], preferred_element_type=F32)
                  + br_ref[...])
        mx = jnp.max(logits, axis=1, keepdims=True)
        ex = jnp.exp(logits - mx)
        aff_s[pl.ds(j * _PB, _PB), :] = ex / jnp.sum(ex, axis=1, keepdims=True)

    @pl.when(j == _NP)
    def _():
        aff = aff_s[...]
        lane = lax.broadcasted_iota(I32, (S, NR), 1)
        m1 = jnp.max(aff, axis=1, keepdims=True)
        i1 = jnp.min(jnp.where(aff == m1, lane, NR), axis=1, keepdims=True)
        aff2 = jnp.where(lane == i1, -1.0, aff)
        m2 = jnp.max(aff2, axis=1, keepdims=True)
        i2 = jnp.min(jnp.where(aff2 == m2, lane, NR), axis=1, keepdims=True)
        tv_ref[...] = jnp.concatenate([m1, m2], axis=1)

        o0 = (lane == i1).astype(F32)
        o1 = (lane == i2).astype(F32)
        oh = jnp.concatenate([o0, o1], axis=0)

        ch = 512
        tril = (lax.broadcasted_iota(I32, (ch, ch), 0) >
                lax.broadcasted_iota(I32, (ch, ch), 1)).astype(F32)
        run = jnp.zeros((1, NR), F32)
        pos_chunks = []
        for c in range(NE // ch):
            oc = oh[c * ch:(c + 1) * ch]
            within = jnp.dot(tril, oc, preferred_element_type=F32)
            pos_chunks.append(within + run)
            run = run + jnp.sum(oc, axis=0, keepdims=True)

        pc = jnp.ceil(run * (1.0 / BLK)) * BLK
        sut = (lax.broadcasted_iota(I32, (NR, NR), 0) <
               lax.broadcasted_iota(I32, (NR, NR), 1)).astype(F32)
        base = jnp.dot(pc, sut, preferred_element_type=F32)

        for c in range(NE // ch):
            oc = oh[c * ch:(c + 1) * ch]
            dst = jnp.sum((pos_chunks[c] + base) * oc, axis=1)
            dest_ref[pl.ds(c * ch, ch)] = dst.astype(I32)

        ends = base + pc
        bio = (lax.broadcasted_iota(I32, (64, NR), 0).astype(F32)
               * float(BLK))
        be = jnp.sum((bio >= ends).astype(F32), axis=1)
        be_ref[...] = jnp.minimum(be, NR - 1).astype(I32)


def _post_call(ctx, wo, bo, x, g2, b2, wr, br):
    row = pl.BlockSpec((_PB, H), lambda j: (jnp.minimum(j, _NP - 1), 0))
    vec = pl.BlockSpec((1, H), lambda j: (0, 0))
    return pl.pallas_call(
        _post_body,
        grid=(_NP + 1,),
        in_specs=[row, pl.BlockSpec((H, H), lambda j: (0, 0)), vec, row,
                  vec, vec, pl.BlockSpec((H, NR), lambda j: (0, 0)),
                  pl.BlockSpec((1, NR), lambda j: (0, 0))],
        out_specs=[row, row, pl.BlockSpec((S, TOPK), lambda j: (0, 0)),
                   pl.BlockSpec((NE,), lambda j: (0,)),
                   pl.BlockSpec((64,), lambda j: (0,))],
        out_shape=[jax.ShapeDtypeStruct((S, H), F32),
                   jax.ShapeDtypeStruct((S, H), F32),
                   jax.ShapeDtypeStruct((S, TOPK), F32),
                   jax.ShapeDtypeStruct((NE,), I32),
                   jax.ShapeDtypeStruct((64,), I32)],
        scratch_shapes=[pltpu.VMEM((S, NR), F32)],
    )(ctx, wo, bo, x, g2, b2, wr, br)


def _moe_body(be_ref, hs_ref, w1_ref, b1_ref, w2_ref, b2_ref, out_ref):
    act = _gelu(jnp.dot(hs_ref[...], w1_ref[0], preferred_element_type=F32)
                + b1_ref[0])
    out_ref[...] = (jnp.dot(act, w2_ref[0], preferred_element_type=F32)
                    + b2_ref[0])


def _moe_call(be, hs, rw1, rb1, rw2, rb2):
    grid_spec = pltpu.PrefetchScalarGridSpec(
        num_scalar_prefetch=1,
        grid=(NBLK,),
        in_specs=[
            pl.BlockSpec((BLK, H), lambda i, be: (i, 0)),
            pl.BlockSpec((1, H, INTER), lambda i, be: (be[i], 0, 0)),
            pl.BlockSpec((1, 1, INTER), lambda i, be: (be[i], 0, 0)),
            pl.BlockSpec((1, INTER, H), lambda i, be: (be[i], 0, 0)),
            pl.BlockSpec((1, 1, H), lambda i, be: (be[i], 0, 0)),
        ],
        out_specs=pl.BlockSpec((BLK, H), lambda i, be: (i, 0)),
    )
    return pl.pallas_call(
        _moe_body,
        grid_spec=grid_spec,
        out_shape=jax.ShapeDtypeStruct((PAD, H), F32),
    )(be, hs, rw1, rb1.reshape(NR, 1, INTER), rw2, rb2.reshape(NR, 1, H))


_TPW = 128


@functools.lru_cache(maxsize=None)
def _sc_kernels():
    mesh = plsc.VectorSubcoreMesh(core_axis_name="c", subcore_axis_name="s")
    scratch = [pltpu.VMEM((_TPW, H), F32),
               pltpu.VMEM((_TPW,), I32),
               pltpu.SemaphoreType.DMA]

    @functools.partial(
        pl.kernel,
        out_type=jax.ShapeDtypeStruct((PAD, H), F32),
        mesh=mesh, scratch_types=scratch)
    def scatter_k(h2_hbm, dest_hbm, hs_hbm, rows_v, idx_v, sem):
        wid = lax.axis_index("s") * 2 + lax.axis_index("c")
        t0 = (wid % 16) * _TPW
        off = (wid // 16) * S + t0
        pltpu.sync_copy(h2_hbm.at[pl.ds(t0, _TPW)], rows_v)
        pltpu.sync_copy(dest_hbm.at[pl.ds(off, _TPW)], idx_v)
        pltpu.async_copy(rows_v, hs_hbm.at[idx_v], sem).wait()

    @functools.partial(
        pl.kernel,
        out_type=jax.ShapeDtypeStruct((NE, H), F32),
        mesh=mesh, scratch_types=scratch)
    def gather_k(rout_hbm, dest_hbm, g_hbm, rows_v, idx_v, sem):
        wid = lax.axis_index("s") * 2 + lax.axis_index("c")
        off = (wid // 16) * S + (wid % 16) * _TPW
        pltpu.sync_copy(dest_hbm.at[pl.ds(off, _TPW)], idx_v)
        pltpu.async_copy(rout_hbm.at[idx_v], rows_v, sem).wait()
        pltpu.sync_copy(rows_v, g_hbm.at[pl.ds(off, _TPW)])

    return scatter_k, gather_k


def _sc_scatter(h2, dest):
    return _sc_kernels()[0](h2, dest)


def _sc_gather(rout, dest):
    return _sc_kernels()[1](rout, dest)


def _final_body(h2_ref, w1_ref, b1_ref, w2_ref, b2_ref, a_ref,
                g0_ref, g1_ref, tv_ref, out_ref):
    h = h2_ref[...]
    acc = (a_ref[...] + h
           + tv_ref[:, 0:1] * g0_ref[...] + tv_ref[:, 1:2] * g1_ref[...])
    for e in range(NSH):
        act = _gelu(jnp.dot(h, w1_ref[e], preferred_element_type=F32)
                    + b1_ref[e:e + 1, :])
        acc = acc + (jnp.dot(act, w2_ref[e], preferred_element_type=F32)
                     + b2_ref[e:e + 1, :])
    out_ref[...] = acc


def _final_call(h2, sw1, sb1, sw2, sb2, a, g, tv):
    blk = 512
    row = pl.BlockSpec((blk, H), lambda i: (i, 0))
    return pl.pallas_call(
        _final_body,
        grid=(S // blk,),
        in_specs=[row,
                  pl.BlockSpec((NSH, H, INTER), lambda i: (0, 0, 0)),
                  pl.BlockSpec((NSH, INTER), lambda i: (0, 0)),
                  pl.BlockSpec((NSH, INTER, H), lambda i: (0, 0, 0)),
                  pl.BlockSpec((NSH, H), lambda i: (0, 0)),
                  row,
                  pl.BlockSpec((blk, H), lambda i: (i, 0)),
                  pl.BlockSpec((blk, H), lambda i: (i + S // blk, 0)),
                  pl.BlockSpec((blk, TOPK), lambda i: (i, 0))],
        out_specs=row,
        out_shape=jax.ShapeDtypeStruct((S, H), F32),
    )(h2, sw1, sb1, sw2, sb2, a, g, g, tv)


def kernel(x, ln1_g, ln1_b, ln2_g, ln2_b, Wq, bq, Wk, bk, Wv, bv, Wo, bo,
           Wr, br, sW1, sb1, sW2, sb2, rW1, rb1, rW2, rb2):
    x2 = x.reshape(S, H)
    r = lambda t: t.reshape(1, -1)
    ctx = _attn_call(x2, r(ln1_g), r(ln1_b), Wq, r(bq), Wk, r(bk), Wv, r(bv))
    a, h2, tv, dest, be = _post_call(ctx, Wo, r(bo), x2,
                                     r(ln2_g), r(ln2_b), Wr, r(br))
    hs = _sc_scatter(h2, dest)
    rout = _moe_call(be, hs, rW1, rb1, rW2, rb2)
    g = _sc_gather(rout, dest)
    out = _final_call(h2, sW1, sb1, sW2, sb2, a, g, tv)
    return out.reshape(1, S, H)


# speedup vs baseline: 1.0110x; 1.0110x over previous
import functools

import jax
import jax.numpy as jnp
from jax import lax
from jax.experimental import pallas as pl
from jax.experimental.pallas import tpu as pltpu
from jax.experimental.pallas import tpu_sc as plsc

S = 2048
H = 768
NH = 12
HD = 64
NR = 14
NSH = 2
TOPK = 2
INTER = 768
BLK = 128
NBLK = 46
PAD = NBLK * BLK
NE = TOPK * S
F32 = jnp.float32
I32 = jnp.int32
_SCALE = 1.0 / (HD ** 0.5)
_SQRT_HALF = 0.7071067811865476


def _ln(h, g, b):
    m = jnp.mean(h, axis=-1, keepdims=True)
    c = h - m
    v = jnp.mean(c * c, axis=-1, keepdims=True)
    return c * lax.rsqrt(v + 1e-5) * g + b


def _gelu(x):
    return 0.5 * x * (1.0 + lax.erf(x * _SQRT_HALF))


_QB = 256
_NQ = S // _QB


def _attn_body(x_ref, g_ref, b_ref, wq_ref, bq_ref, wk_ref, bk_ref,
               wv_ref, bv_ref, o_ref, q_s, k_s, v_s):
    j = pl.program_id(0)

    @pl.when(j < _NQ)
    def _():
        h = _ln(x_ref[...], g_ref[...], b_ref[...])
        i = j * _QB
        q_s[pl.ds(i, _QB), :] = (jnp.dot(h, wq_ref[...],
                                         preferred_element_type=F32)
                                 + bq_ref[...]) * _SCALE
        k_s[pl.ds(i, _QB), :] = (jnp.dot(h, wk_ref[...],
                                         preferred_element_type=F32)
                                 + bk_ref[...])
        v_s[pl.ds(i, _QB), :] = (jnp.dot(h, wv_ref[...],
                                         preferred_element_type=F32)
                                 + bv_ref[...])

    @pl.when(j >= _NQ)
    def _():
        hp = j - _NQ
        c0 = pl.multiple_of(hp * 2 * HD, 2 * HD)
        qp = q_s[:, pl.ds(c0, 2 * HD)]
        kp = k_s[:, pl.ds(c0, 2 * HD)]
        vp = v_s[:, pl.ds(c0, 2 * HD)]
        for t in range(2):
            q = qp[:, t * HD:(t + 1) * HD]
            k = kp[:, t * HD:(t + 1) * HD]
            v = vp[:, t * HD:(t + 1) * HD]
            s = lax.dot_general(q, k, (((1,), (1,)), ((), ())),
                                preferred_element_type=F32)
            p = jnp.exp(s)
            ve = jnp.concatenate([v, jnp.ones((S, 1), F32)], axis=1)
            r = jnp.dot(p, ve, preferred_element_type=F32)
            o_ref[:, t * HD:(t + 1) * HD] = r[:, :HD] / r[:, HD:HD + 1]


def _attn_call(x, g, b, wq, bq, wk, bk, wv, bv):
    hw = 2 * HD
    full = pl.BlockSpec((H, H), lambda j: (0, 0))
    vec = pl.BlockSpec((1, H), lambda j: (0, 0))
    xrow = pl.BlockSpec((_QB, H), lambda j: (jnp.minimum(j, _NQ - 1), 0))
    return pl.pallas_call(
        _attn_body,
        grid=(_NQ + NH // 2,),
        in_specs=[xrow, vec, vec, full, vec, full, vec, full, vec],
        out_specs=pl.BlockSpec((S, hw),
                               lambda j: (0, jnp.maximum(j - _NQ, 0))),
        out_shape=jax.ShapeDtypeStruct((S, H), F32),
        scratch_shapes=[pltpu.VMEM((S, H), F32)] * 3,
    )(x, g, b, wq, bq, wk, bk, wv, bv)


def _post_body(ctx_ref, wo_ref, bo_ref, x_ref, g2_ref, b2_ref, wr_ref, br_ref,
               a_ref, h2_ref, tv_ref, dest_ref, be_ref):
    a = jnp.dot(ctx_ref[...], wo_ref[...], preferred_element_type=F32)
    a = a + bo_ref[...] + x_ref[...]
    a_ref[...] = a
    h2 = _ln(a, g2_ref[...], b2_ref[...])
    h2_ref[...] = h2

    logits = jnp.dot(h2, wr_ref[...], preferred_element_type=F32) + br_ref[...]
    mx = jnp.max(logits, axis=1, keepdims=True)
    ex = jnp.exp(logits - mx)
    aff = ex / jnp.sum(ex, axis=1, keepdims=True)

    lane = lax.broadcasted_iota(I32, (S, NR), 1)
    m1 = jnp.max(aff, axis=1, keepdims=True)
    i1 = jnp.min(jnp.where(aff == m1, lane, NR), axis=1, keepdims=True)
    aff2 = jnp.where(lane == i1, -1.0, aff)
    m2 = jnp.max(aff2, axis=1, keepdims=True)
    i2 = jnp.min(jnp.where(aff2 == m2, lane, NR), axis=1, keepdims=True)
    tv_ref[...] = jnp.concatenate([m1, m2], axis=1)

    o0 = (lane == i1).astype(F32)
    o1 = (lane == i2).astype(F32)
    oh = jnp.concatenate([o0, o1], axis=0)

    ch = 512
    tril = (lax.broadcasted_iota(I32, (ch, ch), 0) >
            lax.broadcasted_iota(I32, (ch, ch), 1)).astype(F32)
    run = jnp.zeros((1, NR), F32)
    pos_chunks = []
    for c in range(NE // ch):
        oc = oh[c * ch:(c + 1) * ch]
        within = jnp.dot(tril, oc, preferred_element_type=F32)
        pos_chunks.append(within + run)
        run = run + jnp.sum(oc, axis=0, keepdims=True)

    pc = jnp.ceil(run * (1.0 / BLK)) * BLK
    sut = (lax.broadcasted_iota(I32, (NR, NR), 0) <
           lax.broadcasted_iota(I32, (NR, NR), 1)).astype(F32)
    base = jnp.dot(pc, sut, preferred_element_type=F32)

    for c in range(NE // ch):
        oc = oh[c * ch:(c + 1) * ch]
        dst = jnp.sum((pos_chunks[c] + base) * oc, axis=1)
        dest_ref[pl.ds(c * ch, ch)] = dst.astype(I32)

    ends = base + pc
    bio = lax.broadcasted_iota(I32, (64, NR), 0).astype(F32) * float(BLK)
    be = jnp.sum((bio >= ends).astype(F32), axis=1)
    be_ref[...] = jnp.minimum(be, NR - 1).astype(I32)


def _post_call(ctx, wo, bo, x, g2, b2, wr, br):
    full = pl.BlockSpec((S, H), lambda: (0, 0))
    vec = pl.BlockSpec((1, H), lambda: (0, 0))
    return pl.pallas_call(
        _post_body,
        in_specs=[full, pl.BlockSpec((H, H), lambda: (0, 0)), vec, full,
                  vec, vec, pl.BlockSpec((H, NR), lambda: (0, 0)),
                  pl.BlockSpec((1, NR), lambda: (0, 0))],
        out_specs=[full, full, pl.BlockSpec((S, TOPK), lambda: (0, 0)),
                   pl.BlockSpec((NE,), lambda: (0,)),
                   pl.BlockSpec((64,), lambda: (0,))],
        out_shape=[jax.ShapeDtypeStruct((S, H), F32),
                   jax.ShapeDtypeStruct((S, H), F32),
                   jax.ShapeDtypeStruct((S, TOPK), F32),
                   jax.ShapeDtypeStruct((NE,), I32),
                   jax.ShapeDtypeStruct((64,), I32)],
    )(ctx, wo, bo, x, g2, b2, wr, br)


def _moe_body(be_ref, hs_ref, w1_ref, b1_ref, w2_ref, b2_ref, out_ref):
    act = _gelu(jnp.dot(hs_ref[...], w1_ref[0], preferred_element_type=F32)
                + b1_ref[0])
    out_ref[...] = (jnp.dot(act, w2_ref[0], preferred_element_type=F32)
                    + b2_ref[0])


def _moe_call(be, hs, rw1, rb1, rw2, rb2):
    grid_spec = pltpu.PrefetchScalarGridSpec(
        num_scalar_prefetch=1,
        grid=(NBLK,),
        in_specs=[
            pl.BlockSpec((BLK, H), lambda i, be: (i, 0)),
            pl.BlockSpec((1, H, INTER), lambda i, be: (be[i], 0, 0)),
            pl.BlockSpec((1, 1, INTER), lambda i, be: (be[i], 0, 0)),
            pl.BlockSpec((1, INTER, H), lambda i, be: (be[i], 0, 0)),
            pl.BlockSpec((1, 1, H), lambda i, be: (be[i], 0, 0)),
        ],
        out_specs=pl.BlockSpec((BLK, H), lambda i, be: (i, 0)),
    )
    return pl.pallas_call(
        _moe_body,
        grid_spec=grid_spec,
        out_shape=jax.ShapeDtypeStruct((PAD, H), F32),
    )(be, hs, rw1, rb1.reshape(NR, 1, INTER), rw2, rb2.reshape(NR, 1, H))


_TPW = 128


@functools.lru_cache(maxsize=None)
def _sc_kernels():
    mesh = plsc.VectorSubcoreMesh(core_axis_name="c", subcore_axis_name="s")
    scratch = [pltpu.VMEM((_TPW, H), F32),
               pltpu.VMEM((_TPW,), I32),
               pltpu.SemaphoreType.DMA]

    @functools.partial(
        pl.kernel,
        out_type=jax.ShapeDtypeStruct((PAD, H), F32),
        mesh=mesh, scratch_types=scratch)
    def scatter_k(h2_hbm, dest_hbm, hs_hbm, rows_v, idx_v, sem):
        wid = lax.axis_index("s") * 2 + lax.axis_index("c")
        t0 = (wid % 16) * _TPW
        off = (wid // 16) * S + t0
        pltpu.sync_copy(h2_hbm.at[pl.ds(t0, _TPW)], rows_v)
        pltpu.sync_copy(dest_hbm.at[pl.ds(off, _TPW)], idx_v)
        pltpu.async_copy(rows_v, hs_hbm.at[idx_v], sem).wait()

    @functools.partial(
        pl.kernel,
        out_type=jax.ShapeDtypeStruct((NE, H), F32),
        mesh=mesh, scratch_types=scratch)
    def gather_k(rout_hbm, dest_hbm, g_hbm, rows_v, idx_v, sem):
        wid = lax.axis_index("s") * 2 + lax.axis_index("c")
        off = (wid // 16) * S + (wid % 16) * _TPW
        pltpu.sync_copy(dest_hbm.at[pl.ds(off, _TPW)], idx_v)
        pltpu.async_copy(rout_hbm.at[idx_v], rows_v, sem).wait()
        pltpu.sync_copy(rows_v, g_hbm.at[pl.ds(off, _TPW)])

    return scatter_k, gather_k


def _sc_scatter(h2, dest):
    return _sc_kernels()[0](h2, dest)


def _sc_gather(rout, dest):
    return _sc_kernels()[1](rout, dest)


def _final_body(h2_ref, w1_ref, b1_ref, w2_ref, b2_ref, a_ref,
                g0_ref, g1_ref, tv_ref, out_ref):
    h = h2_ref[...]
    acc = (a_ref[...] + h
           + tv_ref[:, 0:1] * g0_ref[...] + tv_ref[:, 1:2] * g1_ref[...])
    for e in range(NSH):
        act = _gelu(jnp.dot(h, w1_ref[e], preferred_element_type=F32)
                    + b1_ref[e:e + 1, :])
        acc = acc + (jnp.dot(act, w2_ref[e], preferred_element_type=F32)
                     + b2_ref[e:e + 1, :])
    out_ref[...] = acc


def _final_call(h2, sw1, sb1, sw2, sb2, a, g, tv):
    blk = 512
    row = pl.BlockSpec((blk, H), lambda i: (i, 0))
    return pl.pallas_call(
        _final_body,
        grid=(S // blk,),
        in_specs=[row,
                  pl.BlockSpec((NSH, H, INTER), lambda i: (0, 0, 0)),
                  pl.BlockSpec((NSH, INTER), lambda i: (0, 0)),
                  pl.BlockSpec((NSH, INTER, H), lambda i: (0, 0, 0)),
                  pl.BlockSpec((NSH, H), lambda i: (0, 0)),
                  row,
                  pl.BlockSpec((blk, H), lambda i: (i, 0)),
                  pl.BlockSpec((blk, H), lambda i: (i + S // blk, 0)),
                  pl.BlockSpec((blk, TOPK), lambda i: (i, 0))],
        out_specs=row,
        out_shape=jax.ShapeDtypeStruct((S, H), F32),
    )(h2, sw1, sb1, sw2, sb2, a, g, g, tv)


def kernel(x, ln1_g, ln1_b, ln2_g, ln2_b, Wq, bq, Wk, bk, Wv, bv, Wo, bo,
           Wr, br, sW1, sb1, sW2, sb2, rW1, rb1, rW2, rb2):
    x2 = x.reshape(S, H)
    r = lambda t: t.reshape(1, -1)
    ctx = _attn_call(x2, r(ln1_g), r(ln1_b), Wq, r(bq), Wk, r(bk), Wv, r(bv))
    a, h2, tv, dest, be = _post_call(ctx, Wo, r(bo), x2,
                                     r(ln2_g), r(ln2_b), Wr, r(br))
    hs = _sc_scatter(h2, dest)
    rout = _moe_call(be, hs, rW1, rb1, rW2, rb2)
    g = _sc_gather(rout, dest)
    out = _final_call(h2, sW1, sb1, sW2, sb2, a, g, tv)
    return out.reshape(1, S, H)
